# Initial kernel scaffold; baseline (speedup 1.0000x reference)
import jax
import jax.numpy as jnp
from jax.experimental import pallas as pl


def kernel(ft_a, ft_p, edge_a2p, edge_p2a, W0_ap, W0_pa, W1_ap, W1_pa, Wfc_a, bfc_a, Wfc_p, bfc_p):
    raise NotImplementedError("write your pallas kernel here")


# speedup vs baseline: 3.1943x; 3.1943x over previous
import functools

import jax
import jax.numpy as jnp
from jax import lax
from jax.experimental import pallas as pl
from jax.experimental.pallas import tpu as pltpu
from jax.experimental.pallas import tpu_sc as plsc

N = 10000
E = 320000
NC = 2
NS = 16
NW = NC * NS
CH = 128
NCHUNK = (E // NW + CH - 1) // CH
EPAD = NW * NCHUNK * CH
ACC_ROWS = 10240
ZROWS = 640
DUMMY = N
WB = 624
WB_TAIL = N - NS * WB
HB = 2048
HGRID = EPAD // HB


def _pack_edges(edge):
    dst = edge[0].astype(jnp.int32)
    src = edge[1].astype(jnp.int32)
    pad = EPAD - E
    dst = jnp.concatenate([dst, jnp.full((pad,), DUMMY, jnp.int32)])
    src = jnp.concatenate([src, jnp.zeros((pad,), jnp.int32)])
    return src.reshape(NW, NCHUNK, CH), dst.reshape(NW, NCHUNK, CH)


def _make_agg():
    mesh = plsc.VectorSubcoreMesh(core_axis_name="c", subcore_axis_name="s")

    @functools.partial(
        pl.kernel,
        mesh=mesh,
        out_type=jax.ShapeDtypeStruct((NC, N, 128), jnp.float32),
        scratch_types=[
            pltpu.VMEM((NCHUNK, CH), jnp.int32),
            pltpu.VMEM((NCHUNK, CH), jnp.int32),
            pltpu.VMEM((CH, 128), jnp.float32),
            pltpu.VMEM_SHARED((ACC_ROWS, 128), jnp.float32),
            pltpu.SemaphoreType.DMA,
        ],
    )
    def agg(table_hbm, src_hbm, dst_hbm, zeros_hbm, out_hbm,
            src_v, dst_v, rows_v, acc_s, sem):
        cid = lax.axis_index("c")
        sid = lax.axis_index("s")
        wid = cid * NS + sid
        pltpu.sync_copy(zeros_hbm, acc_s.at[pl.ds(sid * ZROWS, ZROWS)])
        pltpu.sync_copy(src_hbm.at[wid], src_v)
        pltpu.sync_copy(dst_hbm.at[wid], dst_v)
        plsc.subcore_barrier()

        def body(j, carry):
            pltpu.async_copy(table_hbm.at[src_v.at[j]], rows_v, sem).wait()
            pltpu.sync_copy(rows_v, acc_s.at[dst_v.at[j]], add=True)
            return carry

        lax.fori_loop(0, NCHUNK, body, 0)
        plsc.subcore_barrier()
        pltpu.sync_copy(acc_s.at[pl.ds(sid * WB, WB)],
                        out_hbm.at[cid, pl.ds(sid * WB, WB)])

        @pl.when(sid == 0)
        def _():
            pltpu.sync_copy(acc_s.at[pl.ds(NS * WB, WB_TAIL)],
                            out_hbm.at[cid, pl.ds(NS * WB, WB_TAIL)])

    return agg


def _tc_count(dst_l, dst_s):
    def body(l_ref, s_ref, o_ref):
        @pl.when(pl.program_id(0) == 0)
        def _():
            o_ref[...] = jnp.zeros_like(o_ref)

        q = l_ref[0] >> 7
        r = s_ref[0] & 127
        oh_q = (lax.broadcasted_iota(jnp.int32, (80, HB), 0)
                == jnp.broadcast_to(q, (80, HB))).astype(jnp.float32)
        oh_r = (lax.broadcasted_iota(jnp.int32, (HB, 128), 1)
                == jnp.broadcast_to(r, (HB, 128))).astype(jnp.float32)
        o_ref[...] += jnp.dot(oh_q, oh_r, preferred_element_type=jnp.float32)

    return pl.pallas_call(
        body,
        grid=(HGRID,),
        in_specs=[
            pl.BlockSpec((1, 1, HB), lambda i: (i, 0, 0)),
            pl.BlockSpec((1, HB, 1), lambda i: (i, 0, 0)),
        ],
        out_specs=pl.BlockSpec((80, 128), lambda i: (0, 0)),
        out_shape=jax.ShapeDtypeStruct((80, 128), jnp.float32),
    )(dst_l, dst_s)


def _tc_layer1(part, cnt, w0):
    blk = 1000

    def body(p_ref, c_ref, w_ref, emb_ref, dinv_ref):
        d = 1.0 / jnp.maximum(c_ref[...], 1.0)
        mn = (p_ref[0] + p_ref[1]) * d
        emb_ref[...] = jnp.maximum(
            jnp.dot(mn, w_ref[...], preferred_element_type=jnp.float32), 0.0)
        dinv_ref[...] = d

    return pl.pallas_call(
        body,
        grid=(N // blk,),
        in_specs=[
            pl.BlockSpec((NC, blk, 128), lambda i: (0, i, 0)),
            pl.BlockSpec((blk, 1), lambda i: (i, 0)),
            pl.BlockSpec((128, 128), lambda i: (0, 0)),
        ],
        out_specs=[
            pl.BlockSpec((blk, 128), lambda i: (i, 0)),
            pl.BlockSpec((blk, 1), lambda i: (i, 0)),
        ],
        out_shape=[
            jax.ShapeDtypeStruct((N, 128), jnp.float32),
            jax.ShapeDtypeStruct((N, 1), jnp.float32),
        ],
    )(part, cnt, w0)


def _tc_layer2(qpart, dinv, ft, w1, wv, wf, b):
    blk = 1000

    def body(q_ref, d_ref, f_ref, w1_ref, wv_ref, wf_ref, b_ref, o_ref):
        x = (q_ref[0] + q_ref[1]) * d_ref[...]
        v = jnp.maximum(
            jnp.dot(x, w1_ref[...], preferred_element_type=jnp.float32), 0.0)
        o_ref[...] = (
            jnp.dot(v, wv_ref[...], preferred_element_type=jnp.float32)
            + jnp.dot(f_ref[...], wf_ref[...],
                      preferred_element_type=jnp.float32)
            + b_ref[...])

    return pl.pallas_call(
        body,
        grid=(N // blk,),
        in_specs=[
            pl.BlockSpec((NC, blk, 128), lambda i: (0, i, 0)),
            pl.BlockSpec((blk, 1), lambda i: (i, 0)),
            pl.BlockSpec((blk, 128), lambda i: (i, 0)),
            pl.BlockSpec((128, 128), lambda i: (0, 0)),
            pl.BlockSpec((128, 128), lambda i: (0, 0)),
            pl.BlockSpec((128, 128), lambda i: (0, 0)),
            pl.BlockSpec((1, 128), lambda i: (0, 0)),
        ],
        out_specs=pl.BlockSpec((blk, 128), lambda i: (i, 0)),
        out_shape=jax.ShapeDtypeStruct((N, 128), jnp.float32),
    )(qpart, dinv, ft, w1, wv, wf, b)


def _cnt_col(hist):
    return hist.reshape(80 * 128)[:N].reshape(N, 1)


def kernel(ft_a, ft_p, edge_a2p, edge_p2a, W0_ap, W0_pa, W1_ap, W1_pa,
           Wfc_a, bfc_a, Wfc_p, bfc_p):
    src_a, dst_a = _pack_edges(edge_a2p)
    src_p, dst_p = _pack_edges(edge_p2a)
    zeros = jnp.zeros((ZROWS, 128), jnp.float32)
    agg = _make_agg()

    cnt_a = _cnt_col(_tc_count(dst_a.reshape(HGRID, 1, HB),
                               dst_a.reshape(HGRID, HB, 1)))
    cnt_p = _cnt_col(_tc_count(dst_p.reshape(HGRID, 1, HB),
                               dst_p.reshape(HGRID, HB, 1)))

    part_a1 = agg(ft_p, src_a, dst_a, zeros)
    part_p1 = agg(ft_a, src_p, dst_p, zeros)
    emb1_a, dinv_a = _tc_layer1(part_a1, cnt_a, W0_ap)
    emb1_p, dinv_p = _tc_layer1(part_p1, cnt_p, W0_pa)

    part_a2 = agg(emb1_p, src_a, dst_a, zeros)
    part_p2 = agg(emb1_a, src_p, dst_p, zeros)
    out_a = _tc_layer2(part_a2, dinv_a, ft_a, W1_ap,
                       Wfc_a[:128], Wfc_a[128:], bfc_a.reshape(1, 128))
    out_p = _tc_layer2(part_p2, dinv_p, ft_p, W1_pa,
                       Wfc_p[:128], Wfc_p[128:], bfc_p.reshape(1, 128))
    return jnp.concatenate([out_a, out_p], axis=0)
